# baseline (speedup 1.0000x reference)
import jax
import jax.numpy as jnp
from jax.experimental import pallas as pl
from jax.experimental.pallas import tpu as pltpu
from jax.experimental.pallas import tpu_sc as plsc


def _gelu_exact(v):
    return 0.5 * v * (1.0 + jax.lax.erf(v * (2.0 ** -0.5)))


def _bc_to_i32(a):
    n, m = a.shape
    return jax.lax.bitcast_convert_type(
        a.reshape(n, m // 2, 2), jnp.int32)


def _bc_to_bf16(a):
    n, m = a.shape
    return jax.lax.bitcast_convert_type(a, jnp.bfloat16).reshape(n, 2 * m)


def _cumsum_rows(a):
    n = a.shape[0]
    s = 1
    while s < n:
        shifted = jnp.concatenate(
            [jnp.zeros((s, a.shape[1]), a.dtype), a[:-s]], axis=0)
        a = a + shifted
        s *= 2
    return a


B, P, D = 1, 2048, 768
E, K, H = 8, 2, 768
BT = 256
T = (P * K) // BT + E - 1
S = T * BT
SCW = 128


def _gating_body(x_ref, wg_ref, gates_ref, pos0_ref, pos1_ref, eot_ref,
                 loss_ref):
    logits = jnp.dot(x_ref[...], wg_ref[...],
                     preferred_element_type=jnp.float32)
    p = jax.nn.softmax(logits, axis=1)
    iota = jax.lax.broadcasted_iota(jnp.int32, (P, E), 1)
    a1 = jnp.argmax(p, axis=1)
    oh1 = iota == a1[:, None]
    pm = jnp.where(oh1, -1.0, p)
    a2 = jnp.argmax(pm, axis=1)
    oh2 = iota == a2[:, None]
    m1 = jnp.max(p, axis=1, keepdims=True)
    m2 = jnp.max(pm, axis=1, keepdims=True)
    den = m1 + m2 + 1e-6
    g1 = m1 / den
    g2 = m2 / den
    gates_ref[...] = jnp.concatenate([g1, g2], axis=1)

    wmat = jnp.where(oh1, g1, 0.0) + jnp.where(oh2, g2, 0.0)
    imp = jnp.sum(wmat, axis=0)
    load = jnp.sum((wmat > 0).astype(jnp.float32), axis=0)

    def cv_sq(v):
        mean = jnp.sum(v) / E
        var = jnp.sum((v - mean) ** 2) / (E - 1)
        return var / (mean * mean + 1e-10)

    loss_ref[...] = (cv_sq(imp) + cv_sq(load)).reshape(1, 1)

    cnt = oh1.astype(jnp.float32) + oh2.astype(jnp.float32)
    cum = _cumsum_rows(cnt) - cnt
    counts = jnp.sum(cnt, axis=0, keepdims=True)
    padded = jnp.ceil(counts / BT) * BT
    tri = (jax.lax.broadcasted_iota(jnp.int32, (E, E), 0)
           <= jax.lax.broadcasted_iota(jnp.int32, (E, E), 1)
           ).astype(jnp.float32)
    cpad = jnp.dot(padded, tri)
    offs = cpad - padded
    posmat = cum + offs
    pos0 = jnp.sum(jnp.where(oh1, posmat, 0.0), axis=1).astype(jnp.int32)
    pos1 = jnp.sum(jnp.where(oh2, posmat, 0.0), axis=1).astype(jnp.int32)
    pos0_ref[...] = pos0.reshape(1, P)
    pos1_ref[...] = pos1.reshape(1, P)

    tstart = jax.lax.broadcasted_iota(jnp.int32, (T, E), 0).astype(
        jnp.float32) * BT
    eot = jnp.sum((tstart >= cpad).astype(jnp.int32), axis=1)
    total = jnp.max(cpad)
    valid = (tstart[:, 0] < total).astype(jnp.int32)
    eot_ref[...] = jnp.concatenate(
        [jnp.minimum(eot, E - 1).reshape(1, T), valid.reshape(1, T)], axis=0)


def _gating_call(x_flat, w_gate):
    return pl.pallas_call(
        _gating_body,
        out_shape=[
            jax.ShapeDtypeStruct((P, K), jnp.float32),
            jax.ShapeDtypeStruct((1, P), jnp.int32),
            jax.ShapeDtypeStruct((1, P), jnp.int32),
            jax.ShapeDtypeStruct((2, T), jnp.int32),
            jax.ShapeDtypeStruct((1, 1), jnp.float32),
        ],
    )(x_flat, w_gate)


def _dispatch_call(x_flat, pos0, pos1):
    mesh = plsc.VectorSubcoreMesh(core_axis_name="c", subcore_axis_name="s")
    DW = D // 2

    @pl.kernel(out_type=jax.ShapeDtypeStruct((2 * S, DW), jnp.float32),
               mesh=mesh)
    def disp(x_hbm, p0_hbm, p1_hbm, o_hbm):
        def body(x_vmem, p0_vmem, p1_vmem):
            pltpu.sync_copy(x_vmem, o_hbm.at[p0_vmem.at[0]])
            pltpu.sync_copy(x_vmem, o_hbm.at[p1_vmem.at[0]])

        pltpu.emit_pipeline(
            body,
            grid=(2 * P // SCW,),
            in_specs=[
                pl.BlockSpec((SCW, DW), lambda i: (i, 0)),
                pl.BlockSpec((1, SCW), lambda i: (0, i)),
                pl.BlockSpec((1, SCW), lambda i: (0, i)),
            ],
            out_specs=[],
            core_axis_name=("c", "s"),
            dimension_semantics=(pltpu.PARALLEL,),
        )(x_hbm, p0_hbm, p1_hbm)

    return disp(x_flat.reshape(2 * P, DW), pos0, pos1)


def _gmm_body(eot_ref, xs_ref, w1_ref, b1_ref, w2_ref, b2_ref, out_ref):
    t = pl.program_id(0)

    @pl.when(eot_ref[1, t] == 1)
    def _():
        h = jnp.dot(xs_ref[...].astype(jnp.bfloat16), w1_ref[0],
                    preferred_element_type=jnp.float32)
        h = _gelu_exact(h + b1_ref[0])
        o = jnp.dot(h.astype(jnp.bfloat16), w2_ref[0],
                    preferred_element_type=jnp.float32)
        out_ref[...] = o + b2_ref[0]


def _gmm_call(xs, w1b, b1, w2b, b2, eot):
    grid_spec = pltpu.PrefetchScalarGridSpec(
        num_scalar_prefetch=1,
        grid=(T,),
        in_specs=[
            pl.BlockSpec((BT, D), lambda t, eot: (t, 0)),
            pl.BlockSpec((1, D, H), lambda t, eot: (eot[0, t], 0, 0)),
            pl.BlockSpec((1, 1, H), lambda t, eot: (eot[0, t], 0, 0)),
            pl.BlockSpec((1, H, D), lambda t, eot: (eot[0, t], 0, 0)),
            pl.BlockSpec((1, 1, D), lambda t, eot: (eot[0, t], 0, 0)),
        ],
        out_specs=pl.BlockSpec((BT, D), lambda t, eot: (t, 0)),
    )
    return pl.pallas_call(
        _gmm_body,
        grid_spec=grid_spec,
        out_shape=jax.ShapeDtypeStruct((S, D), jnp.float32),
    )(eot, xs, w1b, b1.reshape(E, 1, H), w2b, b2.reshape(E, 1, D))


def _gather_rows(os_view, pos):
    mesh = plsc.VectorSubcoreMesh(core_axis_name="c", subcore_axis_name="s")
    DW = D // 2

    @pl.kernel(out_type=jax.ShapeDtypeStruct((P, DW), jnp.float32),
               mesh=mesh)
    def comb(os_hbm, p_hbm, c_hbm):
        def body(p_vmem, c_vmem):
            pltpu.sync_copy(os_hbm.at[p_vmem.at[0]], c_vmem)

        pltpu.emit_pipeline(
            body,
            grid=(P // SCW,),
            in_specs=[pl.BlockSpec((1, SCW), lambda i: (0, i))],
            out_specs=[pl.BlockSpec((SCW, DW), lambda i: (i, 0))],
            core_axis_name=("c", "s"),
            dimension_semantics=(pltpu.PARALLEL,),
        )(p_hbm, c_hbm)

    return comb(os_view, pos)


def _combine_call(out_sorted, e0, o0, e1, o1):
    osv = out_sorted.reshape(2 * S, D // 2)
    return (_gather_rows(osv, e0), _gather_rows(osv, o0),
            _gather_rows(osv, e1), _gather_rows(osv, o1))


FBT = 256


def _final_body(x_ref, c0l_ref, c0h_ref, c1l_ref, c1h_ref, g_ref,
                gamma_ref, y_ref):
    g0 = g_ref[:, 0][:, None]
    g1 = g_ref[:, 1][:, None]
    c0 = jnp.concatenate([c0l_ref[...], c0h_ref[...]], axis=1)
    c1 = jnp.concatenate([c1l_ref[...], c1h_ref[...]], axis=1)
    y = x_ref[...] + g0 * c0 + g1 * c1
    norm = jnp.sqrt(jnp.sum(y * y, axis=1, keepdims=True))
    y = y / jnp.maximum(norm, 1e-12) * gamma_ref[...] * (float(D) ** 0.5)
    y_ref[...] = _gelu_exact(y)


def _final_call(x_flat, c0l, c0h, c1l, c1h, gates, gamma):
    HW = D // 2
    return pl.pallas_call(
        _final_body,
        grid=(P // FBT,),
        in_specs=[
            pl.BlockSpec((FBT, D), lambda t: (t, 0)),
            pl.BlockSpec((FBT, HW), lambda t: (t, 0)),
            pl.BlockSpec((FBT, HW), lambda t: (t, 0)),
            pl.BlockSpec((FBT, HW), lambda t: (t, 0)),
            pl.BlockSpec((FBT, HW), lambda t: (t, 0)),
            pl.BlockSpec((FBT, K), lambda t: (t, 0)),
            pl.BlockSpec((1, D), lambda t: (0, 0)),
        ],
        out_specs=pl.BlockSpec((FBT, D), lambda t: (t, 0)),
        out_shape=jax.ShapeDtypeStruct((P, D), jnp.float32),
    )(x_flat, c0l, c0h, c1l, c1h, gates, gamma.reshape(1, D))


@jax.jit
def kernel(x, w_gate, W1, b1, W2, b2, gamma):
    x_flat = x.reshape(P, D)
    x16 = x_flat.astype(jnp.bfloat16)
    w1b = W1.astype(jnp.bfloat16)
    w2b = W2.astype(jnp.bfloat16)
    gates, pos0, pos1, eot, loss = _gating_call(x_flat, w_gate)
    q0 = jnp.stack([2 * pos0, 2 * pos0 + 1], axis=2).reshape(1, 2 * P)
    q1 = jnp.stack([2 * pos1, 2 * pos1 + 1], axis=2).reshape(1, 2 * P)
    xs = _dispatch_call(x_flat, q0, q1).reshape(S, D)
    out_sorted = _gmm_call(xs, w1b, b1, w2b, b2, eot)
    e0, o0 = 2 * pos0, 2 * pos0 + 1
    e1, o1 = 2 * pos1, 2 * pos1 + 1
    c0l, c0h, c1l, c1h = _combine_call(out_sorted, e0, o0, e1, o1)
    y = _final_call(x_flat, c0l, c0h, c1l, c1h, gates, gamma)
    return y.reshape(B, P, D), loss[0, 0]


# speedup vs baseline: 1.0382x; 1.0382x over previous
import jax
import jax.numpy as jnp
from jax.experimental import pallas as pl
from jax.experimental.pallas import tpu as pltpu
from jax.experimental.pallas import tpu_sc as plsc


def _gelu_exact(v):
    return 0.5 * v * (1.0 + jax.lax.erf(v * (2.0 ** -0.5)))


def _bc_to_i32(a):
    n, m = a.shape
    return jax.lax.bitcast_convert_type(
        a.reshape(n, m // 2, 2), jnp.int32)


def _bc_to_bf16(a):
    n, m = a.shape
    return jax.lax.bitcast_convert_type(a, jnp.bfloat16).reshape(n, 2 * m)


def _cumsum_rows(a):
    n = a.shape[0]
    s = 1
    while s < n:
        shifted = jnp.concatenate(
            [jnp.zeros((s, a.shape[1]), a.dtype), a[:-s]], axis=0)
        a = a + shifted
        s *= 2
    return a


B, P, D = 1, 2048, 768
E, K, H = 8, 2, 768
BT = 256
T = (P * K) // BT + E - 1
S = T * BT
SCW = 128


def _gating_body(x_ref, wg_ref, gates_ref, pos0_ref, pos1_ref, eot_ref,
                 loss_ref):
    logits = jnp.dot(x_ref[...], wg_ref[...],
                     preferred_element_type=jnp.float32)
    p = jax.nn.softmax(logits, axis=1)
    iota = jax.lax.broadcasted_iota(jnp.int32, (P, E), 1)
    a1 = jnp.argmax(p, axis=1)
    oh1 = iota == a1[:, None]
    pm = jnp.where(oh1, -1.0, p)
    a2 = jnp.argmax(pm, axis=1)
    oh2 = iota == a2[:, None]
    m1 = jnp.max(p, axis=1, keepdims=True)
    m2 = jnp.max(pm, axis=1, keepdims=True)
    den = m1 + m2 + 1e-6
    g1 = m1 / den
    g2 = m2 / den
    gates_ref[...] = jnp.concatenate([g1, g2], axis=1)

    wmat = jnp.where(oh1, g1, 0.0) + jnp.where(oh2, g2, 0.0)
    imp = jnp.sum(wmat, axis=0)
    load = jnp.sum((wmat > 0).astype(jnp.float32), axis=0)

    def cv_sq(v):
        mean = jnp.sum(v) / E
        var = jnp.sum((v - mean) ** 2) / (E - 1)
        return var / (mean * mean + 1e-10)

    loss_ref[...] = (cv_sq(imp) + cv_sq(load)).reshape(1, 1)

    cnt = oh1.astype(jnp.float32) + oh2.astype(jnp.float32)
    cum = _cumsum_rows(cnt) - cnt
    counts = jnp.sum(cnt, axis=0, keepdims=True)
    padded = jnp.ceil(counts / BT) * BT
    tri = (jax.lax.broadcasted_iota(jnp.int32, (E, E), 0)
           <= jax.lax.broadcasted_iota(jnp.int32, (E, E), 1)
           ).astype(jnp.float32)
    cpad = jnp.dot(padded, tri)
    offs = cpad - padded
    posmat = cum + offs
    pos0 = jnp.sum(jnp.where(oh1, posmat, 0.0), axis=1).astype(jnp.int32)
    pos1 = jnp.sum(jnp.where(oh2, posmat, 0.0), axis=1).astype(jnp.int32)
    pos0_ref[...] = pos0.reshape(1, P)
    pos1_ref[...] = pos1.reshape(1, P)

    tstart = jax.lax.broadcasted_iota(jnp.int32, (T, E), 0).astype(
        jnp.float32) * BT
    eot = jnp.sum((tstart >= cpad).astype(jnp.int32), axis=1)
    total = jnp.max(cpad)
    valid = (tstart[:, 0] < total).astype(jnp.int32)
    eot_ref[...] = jnp.concatenate(
        [jnp.minimum(eot, E - 1).reshape(1, T), valid.reshape(1, T)], axis=0)


def _gating_call(x_flat, w_gate):
    return pl.pallas_call(
        _gating_body,
        out_shape=[
            jax.ShapeDtypeStruct((P, K), jnp.float32),
            jax.ShapeDtypeStruct((1, P), jnp.int32),
            jax.ShapeDtypeStruct((1, P), jnp.int32),
            jax.ShapeDtypeStruct((2, T), jnp.int32),
            jax.ShapeDtypeStruct((1, 1), jnp.float32),
        ],
    )(x_flat, w_gate)


def _dispatch_call(x_flat, pos0, pos1):
    mesh = plsc.VectorSubcoreMesh(core_axis_name="c", subcore_axis_name="s")
    DW = D // 2

    @pl.kernel(out_type=jax.ShapeDtypeStruct((2 * S, DW), jnp.float32),
               mesh=mesh)
    def disp(x_hbm, p0_hbm, p1_hbm, o_hbm):
        def body(x_vmem, p0_vmem, p1_vmem):
            pltpu.sync_copy(x_vmem, o_hbm.at[p0_vmem.at[0]])
            pltpu.sync_copy(x_vmem, o_hbm.at[p1_vmem.at[0]])

        pltpu.emit_pipeline(
            body,
            grid=(2 * P // SCW,),
            in_specs=[
                pl.BlockSpec((SCW, DW), lambda i: (i, 0)),
                pl.BlockSpec((1, SCW), lambda i: (0, i)),
                pl.BlockSpec((1, SCW), lambda i: (0, i)),
            ],
            out_specs=[],
            core_axis_name=("c", "s"),
            dimension_semantics=(pltpu.PARALLEL,),
        )(x_hbm, p0_hbm, p1_hbm)

    return disp(x_flat.reshape(2 * P, DW), pos0, pos1)


def _gmm_body(eot_ref, xs_ref, w1_ref, b1_ref, w2_ref, b2_ref, out_ref):
    t = pl.program_id(0)

    @pl.when(eot_ref[1, t] == 1)
    def _():
        h = jnp.dot(xs_ref[...].astype(jnp.bfloat16), w1_ref[0],
                    preferred_element_type=jnp.float32)
        h = _gelu_exact(h + b1_ref[0])
        o = jnp.dot(h.astype(jnp.bfloat16), w2_ref[0],
                    preferred_element_type=jnp.float32)
        out_ref[...] = o + b2_ref[0]


def _gmm_call(xs, w1b, b1, w2b, b2, eot):
    grid_spec = pltpu.PrefetchScalarGridSpec(
        num_scalar_prefetch=1,
        grid=(T,),
        in_specs=[
            pl.BlockSpec((BT, D), lambda t, eot: (t, 0)),
            pl.BlockSpec((1, D, H), lambda t, eot: (eot[0, t], 0, 0)),
            pl.BlockSpec((1, 1, H), lambda t, eot: (eot[0, t], 0, 0)),
            pl.BlockSpec((1, H, D), lambda t, eot: (eot[0, t], 0, 0)),
            pl.BlockSpec((1, 1, D), lambda t, eot: (eot[0, t], 0, 0)),
        ],
        out_specs=pl.BlockSpec((BT, D), lambda t, eot: (t, 0)),
    )
    return pl.pallas_call(
        _gmm_body,
        grid_spec=grid_spec,
        out_shape=jax.ShapeDtypeStruct((S, D), jnp.float32),
    )(eot, xs, w1b, b1.reshape(E, 1, H), w2b, b2.reshape(E, 1, D))


def _gather_rows(out_sorted, pos):
    mesh = plsc.VectorSubcoreMesh(core_axis_name="c", subcore_axis_name="s")
    DW = D // 2

    @pl.kernel(out_type=jax.ShapeDtypeStruct((2 * P, DW), jnp.float32),
               mesh=mesh)
    def comb(os_hbm, p_hbm, c_hbm):
        def body(p_vmem, c_vmem):
            pltpu.sync_copy(os_hbm.at[p_vmem.at[0]], c_vmem)

        pltpu.emit_pipeline(
            body,
            grid=(2 * P // SCW,),
            in_specs=[pl.BlockSpec((1, SCW), lambda i: (0, i))],
            out_specs=[pl.BlockSpec((SCW, DW), lambda i: (i, 0))],
            core_axis_name=("c", "s"),
            dimension_semantics=(pltpu.PARALLEL,),
        )(p_hbm, c_hbm)

    return comb(out_sorted.reshape(2 * S, DW), pos).reshape(P, D)


def _combine_call(out_sorted, q0, q1):
    return _gather_rows(out_sorted, q0), _gather_rows(out_sorted, q1)


FBT = 256


def _final_body(x_ref, c0_ref, c1_ref, g_ref, gamma_ref, y_ref):
    g0 = g_ref[:, 0][:, None]
    g1 = g_ref[:, 1][:, None]
    y = x_ref[...] + g0 * c0_ref[...] + g1 * c1_ref[...]
    norm = jnp.sqrt(jnp.sum(y * y, axis=1, keepdims=True))
    y = y / jnp.maximum(norm, 1e-12) * gamma_ref[...] * (float(D) ** 0.5)
    y_ref[...] = _gelu_exact(y)


def _final_call(x_flat, c0, c1, gates, gamma):
    return pl.pallas_call(
        _final_body,
        grid=(P // FBT,),
        in_specs=[
            pl.BlockSpec((FBT, D), lambda t: (t, 0)),
            pl.BlockSpec((FBT, D), lambda t: (t, 0)),
            pl.BlockSpec((FBT, D), lambda t: (t, 0)),
            pl.BlockSpec((FBT, K), lambda t: (t, 0)),
            pl.BlockSpec((1, D), lambda t: (0, 0)),
        ],
        out_specs=pl.BlockSpec((FBT, D), lambda t: (t, 0)),
        out_shape=jax.ShapeDtypeStruct((P, D), jnp.float32),
    )(x_flat, c0, c1, gates, gamma.reshape(1, D))


@jax.jit
def kernel(x, w_gate, W1, b1, W2, b2, gamma):
    x_flat = x.reshape(P, D)
    x16 = x_flat.astype(jnp.bfloat16)
    w1b = W1.astype(jnp.bfloat16)
    w2b = W2.astype(jnp.bfloat16)
    gates, pos0, pos1, eot, loss = _gating_call(x_flat, w_gate)
    q0 = jnp.stack([2 * pos0, 2 * pos0 + 1], axis=2).reshape(1, 2 * P)
    q1 = jnp.stack([2 * pos1, 2 * pos1 + 1], axis=2).reshape(1, 2 * P)
    xs = _dispatch_call(x_flat, q0, q1).reshape(S, D)
    out_sorted = _gmm_call(xs, w1b, b1, w2b, b2, eot)
    c0, c1 = _combine_call(out_sorted, q0, q1)
    y = _final_call(x_flat, c0, c1, gates, gamma)
    return y.reshape(B, P, D), loss[0, 0]
